# Initial kernel scaffold; baseline (speedup 1.0000x reference)
import jax
import jax.numpy as jnp
from jax.experimental import pallas as pl


def kernel(x, edge_index, weights, bias):
    raise NotImplementedError("write your pallas kernel here")


# speedup vs baseline: 161.0646x; 161.0646x over previous
import functools

import jax
import jax.numpy as jnp
from jax import lax
from jax.experimental import pallas as pl
from jax.experimental.pallas import tpu as pltpu
from jax.experimental.pallas import tpu_sc as plsc

INPUT_DIM = 4096
HIDDEN_DIMS = [8192, 8192]
OUTPUT_DIM = 4096
FANIN = 64
LAYER_DIMS = [INPUT_DIM] + HIDDEN_DIMS + [OUTPUT_DIM]
LAYER_INDICES = [0]
for _d in LAYER_DIMS:
    LAYER_INDICES.append(LAYER_INDICES[-1] + _d)

NC = 2
NS = 16
NW = NC * NS
LANES = 16


def _make_layer_kernel(n, prev_n, prev_start):
    npw = n // NW
    epw = npw * FANIN
    mesh = plsc.VectorSubcoreMesh(core_axis_name="c", subcore_axis_name="s")

    @functools.partial(
        pl.kernel,
        out_type=jax.ShapeDtypeStruct((n,), jnp.float32),
        mesh=mesh,
        compiler_params=pltpu.CompilerParams(needs_layout_passes=False),
        scratch_types=[
            pltpu.VMEM((prev_n,), jnp.float32),
            pltpu.VMEM((epw,), jnp.int32),
            pltpu.VMEM((epw,), jnp.float32),
            pltpu.VMEM((npw,), jnp.float32),
            pltpu.VMEM((npw,), jnp.float32),
        ],
    )
    def layer(prev_hbm, src_hbm, w_hbm, b_hbm, out_hbm,
              acts_v, src_v, w_v, b_v, out_v):
        wid = lax.axis_index("s") * NC + lax.axis_index("c")
        e_base = wid * epw
        n_base = wid * npw
        pltpu.sync_copy(prev_hbm, acts_v)
        pltpu.sync_copy(src_hbm.at[pl.ds(e_base, epw)], src_v)
        pltpu.sync_copy(w_hbm.at[pl.ds(e_base, epw)], w_v)
        pltpu.sync_copy(b_hbm.at[pl.ds(n_base, npw)], b_v)

        lane = lax.iota(jnp.int32, LANES)

        def group(g, _):
            pos0 = (g * LANES + lane) * FANIN

            def j_body(j, acc):
                pos = pos0 + j
                s = plsc.load_gather(src_v, [pos])
                w = plsc.load_gather(w_v, [pos])
                a = plsc.load_gather(acts_v, [s - prev_start])
                return acc + a * w

            acc = lax.fori_loop(0, FANIN, j_body,
                                jnp.zeros((LANES,), jnp.float32))
            b = b_v[pl.ds(g * LANES, LANES)]
            out_v[pl.ds(g * LANES, LANES)] = jnp.maximum(acc + b, 0.0)
            return 0

        lax.fori_loop(0, npw // LANES, group, 0)
        pltpu.sync_copy(out_v, out_hbm.at[pl.ds(n_base, npw)])

    return layer


_LAYER_KERNELS = []
for _i in range(1, len(LAYER_DIMS)):
    _LAYER_KERNELS.append(
        _make_layer_kernel(LAYER_DIMS[_i], LAYER_DIMS[_i - 1],
                           LAYER_INDICES[_i - 1]))


def kernel(x, edge_index, weights, bias):
    src = edge_index[0]
    acts = x
    offset = 0
    for i in range(1, len(LAYER_DIMS)):
        n = LAYER_DIMS[i]
        n_e = n * FANIN
        cur_start = LAYER_INDICES[i]
        acts = _LAYER_KERNELS[i - 1](
            acts,
            src[offset:offset + n_e],
            weights[offset:offset + n_e],
            bias[cur_start - INPUT_DIM:cur_start - INPUT_DIM + n],
        )
        offset += n_e
    return acts
